# Initial kernel scaffold; baseline (speedup 1.0000x reference)
import jax
import jax.numpy as jnp
from jax.experimental import pallas as pl


def kernel(severity_ids, weight):
    raise NotImplementedError("write your pallas kernel here")


# speedup vs baseline: 1.0934x; 1.0934x over previous
import jax
import jax.numpy as jnp
from jax import lax
from jax.experimental import pallas as pl
from jax.experimental.pallas import tpu as pltpu
from jax.experimental.pallas import tpu_sc as plsc

EMBED_DIM = 32
NUM_IDS = 16384 * 50
NUM_WORKERS = 32
ROWS_PER_W = NUM_IDS // NUM_WORKERS
CHUNK = 1024
NCHUNK = ROWS_PER_W // CHUNK


def _gather_body(ids_hbm, table_hbm, out_hbm, idx_v, rows_v, sem):
    wid = lax.axis_index("s") * 2 + lax.axis_index("c")
    base = wid * ROWS_PER_W

    def body(i, carry):
        off = base + i * CHUNK
        pltpu.sync_copy(ids_hbm.at[pl.ds(off, CHUNK)], idx_v)
        pltpu.async_copy(table_hbm.at[idx_v], rows_v, sem).wait()
        pltpu.sync_copy(rows_v, out_hbm.at[pl.ds(off, CHUNK)])
        return carry

    lax.fori_loop(0, NCHUNK, body, 0)


@jax.jit
def _run(ids_flat, weight):
    mesh = plsc.VectorSubcoreMesh(core_axis_name="c", subcore_axis_name="s")
    f = pl.kernel(
        _gather_body,
        out_type=jax.ShapeDtypeStruct((NUM_IDS, EMBED_DIM), jnp.float32),
        mesh=mesh,
        scratch_types=[
            pltpu.VMEM((CHUNK,), jnp.int32),
            pltpu.VMEM((CHUNK, EMBED_DIM), jnp.float32),
            pltpu.SemaphoreType.DMA,
        ],
        compiler_params=pltpu.CompilerParams(use_tc_tiling_on_sc=False),
    )
    return f(ids_flat, weight)


def kernel(severity_ids, weight):
    ids_flat = severity_ids.reshape(-1).astype(jnp.int32)
    out = _run(ids_flat, weight)
    return out.reshape(severity_ids.shape + (EMBED_DIM,))


# speedup vs baseline: 1.1125x; 1.0174x over previous
import jax
import jax.numpy as jnp
from jax import lax
from jax.experimental import pallas as pl
from jax.experimental.pallas import tpu as pltpu
from jax.experimental.pallas import tpu_sc as plsc

EMBED_DIM = 32
NUM_IDS = 16384 * 50
NUM_WORKERS = 32
ROWS_PER_W = NUM_IDS // NUM_WORKERS
CHUNK = 1024
NCHUNK = ROWS_PER_W // CHUNK
NBUF = 3


def _gather_body(ids_hbm, table_hbm, out_hbm, idx_v, rows_v, sem_g, sem_s):
    wid = lax.axis_index("s") * 2 + lax.axis_index("c")
    base = wid * ROWS_PER_W

    pltpu.sync_copy(ids_hbm.at[pl.ds(base, ROWS_PER_W)], idx_v)

    gathers, stores = {}, {}

    def start_gather(i):
        b = i % NBUF
        gathers[i] = pltpu.async_copy(
            table_hbm.at[idx_v.at[pl.ds(i * CHUNK, CHUNK)]],
            rows_v[b], sem_g[b])

    def start_store(i):
        b = i % NBUF
        stores[i] = pltpu.async_copy(
            rows_v[b], out_hbm.at[pl.ds(base + i * CHUNK, CHUNK)], sem_s[b])

    for i in range(NCHUNK):
        if i >= NBUF:
            stores[i - NBUF].wait()
        start_gather(i)
        if i >= 1:
            gathers[i - 1].wait()
            start_store(i - 1)
    gathers[NCHUNK - 1].wait()
    start_store(NCHUNK - 1)
    for i in range(max(0, NCHUNK - NBUF), NCHUNK):
        stores[i].wait()


@jax.jit
def _run(ids_flat, weight):
    mesh = plsc.VectorSubcoreMesh(core_axis_name="c", subcore_axis_name="s")
    f = pl.kernel(
        _gather_body,
        out_type=jax.ShapeDtypeStruct((NUM_IDS, EMBED_DIM), jnp.float32),
        mesh=mesh,
        scratch_types=[
            pltpu.VMEM((ROWS_PER_W,), jnp.int32),
            [pltpu.VMEM((CHUNK, EMBED_DIM), jnp.float32) for _ in range(NBUF)],
            [pltpu.SemaphoreType.DMA for _ in range(NBUF)],
            [pltpu.SemaphoreType.DMA for _ in range(NBUF)],
        ],
        compiler_params=pltpu.CompilerParams(use_tc_tiling_on_sc=False),
    )
    return f(ids_flat, weight)


def kernel(severity_ids, weight):
    ids_flat = severity_ids.reshape(-1).astype(jnp.int32)
    out = _run(ids_flat, weight)
    return out.reshape(severity_ids.shape + (EMBED_DIM,))


# speedup vs baseline: 1.7580x; 1.5803x over previous
import jax
import jax.numpy as jnp
from jax import lax
from jax.experimental import pallas as pl
from jax.experimental.pallas import tpu as pltpu
from jax.experimental.pallas import tpu_sc as plsc

B, S, D = 16384, 50, 32
NUM_WORKERS = 32
BPW = B // NUM_WORKERS
G = 8
NGROUP = BPW // G
NSLOT = 2


def _gather_body(ids_hbm, table_hbm, out_hbm, idx_v, rows_v, sem_g, sem_s):
    wid = lax.axis_index("s") * 2 + lax.axis_index("c")
    base = wid * BPW

    pltpu.sync_copy(ids_hbm.at[pl.ds(base * S, BPW * S)], idx_v)

    def group_pair(gg, carry):
        for h in range(NSLOT):
            g = NSLOT * gg + h

            @pl.when(gg >= 1)
            def _wait_prev(h=h):
                for k in range(G):
                    pltpu.make_async_copy(
                        rows_v[h].at[pl.ds(k * S, S), :],
                        out_hbm.at[base], sem_s[h]).wait()

            gather = pltpu.async_copy(
                table_hbm.at[idx_v.at[pl.ds(g * G * S, G * S)]],
                rows_v[h], sem_g[h])
            gather.wait()
            for k in range(G):
                pltpu.async_copy(
                    rows_v[h].at[pl.ds(k * S, S), :],
                    out_hbm.at[base + g * G + k], sem_s[h])
        return carry

    lax.fori_loop(0, NGROUP // NSLOT, group_pair, 0)
    for h in range(NSLOT):
        for k in range(G):
            pltpu.make_async_copy(
                rows_v[h].at[pl.ds(k * S, S), :],
                out_hbm.at[base], sem_s[h]).wait()


@jax.jit
def _run(ids_flat, weight):
    mesh = plsc.VectorSubcoreMesh(core_axis_name="c", subcore_axis_name="s")
    f = pl.kernel(
        _gather_body,
        out_type=jax.ShapeDtypeStruct((B, S, D), jnp.float32),
        mesh=mesh,
        scratch_types=[
            pltpu.VMEM((BPW * S,), jnp.int32),
            [pltpu.VMEM((G * S, D), jnp.float32) for _ in range(NSLOT)],
            [pltpu.SemaphoreType.DMA for _ in range(NSLOT)],
            [pltpu.SemaphoreType.DMA for _ in range(NSLOT)],
        ],
        compiler_params=pltpu.CompilerParams(use_tc_tiling_on_sc=False),
    )
    return f(ids_flat, weight)


def kernel(severity_ids, weight):
    ids_flat = severity_ids.reshape(-1).astype(jnp.int32)
    return _run(ids_flat, weight)


# speedup vs baseline: 1.7916x; 1.0191x over previous
import jax
import jax.numpy as jnp
from jax import lax
from jax.experimental import pallas as pl
from jax.experimental.pallas import tpu as pltpu
from jax.experimental.pallas import tpu_sc as plsc

B, S, D = 16384, 50, 32
NSPLIT = 2
BH = B // NSPLIT
NUM_WORKERS = 32
BPW = BH // NUM_WORKERS
G = 8
NGROUP = BPW // G
NSLOT = 2


def _gather_body(ids_hbm, table_hbm, out_hbm, idx_v, rows_v, sem_g, sem_s):
    wid = lax.axis_index("s") * 2 + lax.axis_index("c")
    base = wid * BPW

    pltpu.sync_copy(ids_hbm.at[pl.ds(base * S, BPW * S)], idx_v)

    def group_pair(gg, carry):
        for h in range(NSLOT):
            g = NSLOT * gg + h

            @pl.when(gg >= 1)
            def _wait_prev(h=h):
                for k in range(G):
                    pltpu.make_async_copy(
                        rows_v[h].at[pl.ds(k * S, S), :],
                        out_hbm.at[base], sem_s[h]).wait()

            gather = pltpu.async_copy(
                table_hbm.at[idx_v.at[pl.ds(g * G * S, G * S)]],
                rows_v[h], sem_g[h])
            gather.wait()
            for k in range(G):
                pltpu.async_copy(
                    rows_v[h].at[pl.ds(k * S, S), :],
                    out_hbm.at[base + g * G + k], sem_s[h])
        return carry

    lax.fori_loop(0, NGROUP // NSLOT, group_pair, 0)
    for h in range(NSLOT):
        for k in range(G):
            pltpu.make_async_copy(
                rows_v[h].at[pl.ds(k * S, S), :],
                out_hbm.at[base], sem_s[h]).wait()


@jax.jit
def _run(ids_flat, weight):
    mesh = plsc.VectorSubcoreMesh(core_axis_name="c", subcore_axis_name="s")
    f = pl.kernel(
        _gather_body,
        out_type=jax.ShapeDtypeStruct((BH, S, D), jnp.float32),
        mesh=mesh,
        scratch_types=[
            pltpu.VMEM((BPW * S,), jnp.int32),
            [pltpu.VMEM((G * S, D), jnp.float32) for _ in range(NSLOT)],
            [pltpu.SemaphoreType.DMA for _ in range(NSLOT)],
            [pltpu.SemaphoreType.DMA for _ in range(NSLOT)],
        ],
        compiler_params=pltpu.CompilerParams(use_tc_tiling_on_sc=False),
    )
    halves = [f(lax.dynamic_slice_in_dim(ids_flat, i * BH * S, BH * S), weight)
              for i in range(NSPLIT)]
    return jnp.concatenate(halves, axis=0)


def kernel(severity_ids, weight):
    ids_flat = severity_ids.reshape(-1).astype(jnp.int32)
    return _run(ids_flat, weight)


# speedup vs baseline: 1.8552x; 1.0355x over previous
import jax
import jax.numpy as jnp
from jax import lax
from jax.experimental import pallas as pl
from jax.experimental.pallas import tpu as pltpu
from jax.experimental.pallas import tpu_sc as plsc

B, S, D = 16384, 50, 32
NSPLIT = 4
BH = B // NSPLIT
NUM_WORKERS = 32
BPW = BH // NUM_WORKERS
G = 8
NGROUP = BPW // G
NSLOT = 2


def _gather_body(ids_hbm, table_hbm, out_hbm, idx_v, rows_v, sem_g, sem_s):
    wid = lax.axis_index("s") * 2 + lax.axis_index("c")
    base = wid * BPW

    pltpu.sync_copy(ids_hbm.at[pl.ds(base * S, BPW * S)], idx_v)

    def group_pair(gg, carry):
        for h in range(NSLOT):
            g = NSLOT * gg + h

            @pl.when(gg >= 1)
            def _wait_prev(h=h):
                for k in range(G):
                    pltpu.make_async_copy(
                        rows_v[h].at[pl.ds(k * S, S), :],
                        out_hbm.at[base], sem_s[h]).wait()

            gather = pltpu.async_copy(
                table_hbm.at[idx_v.at[pl.ds(g * G * S, G * S)]],
                rows_v[h], sem_g[h])
            gather.wait()
            for k in range(G):
                pltpu.async_copy(
                    rows_v[h].at[pl.ds(k * S, S), :],
                    out_hbm.at[base + g * G + k], sem_s[h])
        return carry

    lax.fori_loop(0, NGROUP // NSLOT, group_pair, 0)
    for h in range(NSLOT):
        for k in range(G):
            pltpu.make_async_copy(
                rows_v[h].at[pl.ds(k * S, S), :],
                out_hbm.at[base], sem_s[h]).wait()


@jax.jit
def _run(ids_flat, weight):
    mesh = plsc.VectorSubcoreMesh(core_axis_name="c", subcore_axis_name="s")
    f = pl.kernel(
        _gather_body,
        out_type=jax.ShapeDtypeStruct((BH, S, D), jnp.float32),
        mesh=mesh,
        scratch_types=[
            pltpu.VMEM((BPW * S,), jnp.int32),
            [pltpu.VMEM((G * S, D), jnp.float32) for _ in range(NSLOT)],
            [pltpu.SemaphoreType.DMA for _ in range(NSLOT)],
            [pltpu.SemaphoreType.DMA for _ in range(NSLOT)],
        ],
        compiler_params=pltpu.CompilerParams(use_tc_tiling_on_sc=False),
    )
    halves = [f(lax.dynamic_slice_in_dim(ids_flat, i * BH * S, BH * S), weight)
              for i in range(NSPLIT)]
    return jnp.concatenate(halves, axis=0)


def kernel(severity_ids, weight):
    ids_flat = severity_ids.reshape(-1).astype(jnp.int32)
    return _run(ids_flat, weight)
